# Initial kernel scaffold; baseline (speedup 1.0000x reference)
import jax
import jax.numpy as jnp
from jax.experimental import pallas as pl


def kernel(point_feature_, point_coord_, voxel_label_):
    raise NotImplementedError("write your pallas kernel here")


# speedup vs baseline: 1.6699x; 1.6699x over previous
import functools

import jax
import jax.numpy as jnp
import numpy as np
from jax import lax
from jax.experimental import pallas as pl
from jax.experimental.pallas import tpu as pltpu
from jax.experimental.pallas import tpu_sc as plsc

BATCH = 4
RADIUS = 240
ANGLE = 180
HEIGHT = 32
RADIUS_KEEP = 120
N_POINTS = 480000
CUT_NUM = 8

NC = 2
NS = 16
L = 16
NW = NC * NS

PTS_PER_W = N_POINTS // NW
NGRP = -(-PTS_PER_W // L)
PBUF = NGRP * L


def _build_constants():
    perms = np.array(
        [[1, 3, 0, 2], [2, 0, 3, 1], [0, 1, 2, 3], [3, 2, 0, 1],
         [1, 3, 2, 0], [3, 1, 2, 0], [1, 0, 3, 2], [0, 2, 1, 3]],
        dtype=np.int32,
    )
    inv = np.argsort(perms, axis=1)
    lut = np.ascontiguousarray(inv.reshape(-1).astype(np.int32))
    packed = 0
    for k, v in enumerate(perms.reshape(-1)):
        packed |= int(v) << (2 * k)
    lo = np.uint32(packed & 0xFFFFFFFF).astype(np.int32)
    hi = np.uint32((packed >> 32) & 0xFFFFFFFF).astype(np.int32)
    return lut, int(lo), int(hi)


_LUT_NP, _PERMS_LO, _PERMS_HI = _build_constants()


def _points_body(c0_hbm, c1_hbm, c2_hbm, lut_hbm, out_hbm,
                 b0, b1, b2, ob, lut_v, sem0, sem1, sem2):
    wid = lax.axis_index("s") * NC + lax.axis_index("c")
    base = wid * PTS_PER_W

    cp0 = pltpu.async_copy(c0_hbm.at[pl.ds(base, PTS_PER_W)],
                           b0.at[pl.ds(0, PTS_PER_W)], sem0)
    cp1 = pltpu.async_copy(c1_hbm.at[pl.ds(base, PTS_PER_W)],
                           b1.at[pl.ds(0, PTS_PER_W)], sem1)
    cp2 = pltpu.async_copy(c2_hbm.at[pl.ds(base, PTS_PER_W)],
                           b2.at[pl.ds(0, PTS_PER_W)], sem2)
    pltpu.sync_copy(lut_hbm, lut_v)
    cp0.wait()
    cp1.wait()
    cp2.wait()

    @pl.loop(0, NGRP)
    def _grp(g):
        off = pl.multiple_of(g * L, L)
        c0 = b0[pl.ds(off, L)]
        r = b1[pl.ds(off, L)]
        ang = b2[pl.ds(off, L)]
        sec = ((2 * ang + 1) * 1457) >> 16
        li = (sec * 4 + c0) & 31
        val = plsc.load_gather(lut_v, [li])
        ob[pl.ds(off, L)] = jnp.where(r >= RADIUS_KEEP, val, c0)

    pltpu.sync_copy(ob.at[pl.ds(0, PTS_PER_W)],
                    out_hbm.at[pl.ds(base, PTS_PER_W)])


_points_sc = functools.partial(
    pl.kernel,
    out_type=jax.ShapeDtypeStruct((N_POINTS,), jnp.int32),
    mesh=plsc.VectorSubcoreMesh(
        core_axis_name="c", subcore_axis_name="s",
        num_cores=NC, num_subcores=NS,
    ),
    compiler_params=pltpu.CompilerParams(needs_layout_passes=False),
    scratch_types=[
        pltpu.VMEM((PBUF,), jnp.int32),
        pltpu.VMEM((PBUF,), jnp.int32),
        pltpu.VMEM((PBUF,), jnp.int32),
        pltpu.VMEM((PBUF,), jnp.int32),
        pltpu.VMEM((32,), jnp.int32),
        pltpu.SemaphoreType.DMA,
        pltpu.SemaphoreType.DMA,
        pltpu.SemaphoreType.DMA,
    ],
)(_points_body)


def _vox_body(own_ref, perm_ref, out_ref):
    r_iota = lax.broadcasted_iota(jnp.int32, (1, 1, HEIGHT, RADIUS), 3)
    out_ref[...] = jnp.where(r_iota < RADIUS_KEEP, own_ref[...], perm_ref[...])


def _src_index_map(b, a):
    sec = ((2 * a + 1) * 1457) >> 16
    idx = sec * 4 + b
    word = jnp.where(idx < 16,
                     jnp.int32(_PERMS_LO) >> (2 * idx),
                     jnp.int32(_PERMS_HI) >> (2 * (idx - 16)))
    return word & 3, a, 0, 0


_vox_call = pl.pallas_call(
    _vox_body,
    grid=(BATCH, ANGLE),
    in_specs=[
        pl.BlockSpec((1, 1, HEIGHT, RADIUS), lambda b, a: (b, a, 0, 0)),
        pl.BlockSpec((1, 1, HEIGHT, RADIUS), _src_index_map),
    ],
    out_specs=pl.BlockSpec((1, 1, HEIGHT, RADIUS), lambda b, a: (b, a, 0, 0)),
    out_shape=jax.ShapeDtypeStruct((BATCH, ANGLE, HEIGHT, RADIUS), jnp.int32),
)


def kernel(point_feature_, point_coord_, voxel_label_):
    c0 = point_coord_[:, 0]
    c1 = point_coord_[:, 1]
    c2 = point_coord_[:, 2]
    newc0 = _points_sc(c0, c1, c2, jnp.asarray(_LUT_NP))
    pc_out = jnp.stack([newc0, c1, c2], axis=1)

    vt = jnp.transpose(voxel_label_, (0, 2, 3, 1))
    vo = _vox_call(vt, vt)
    vl_out = jnp.transpose(vo, (0, 3, 1, 2))

    return point_feature_, pc_out, vl_out
